# baseline (speedup 1.0000x reference)
import dataclasses
import functools

import jax
import jax.numpy as jnp
from jax import lax
from jax.experimental import pallas as pl
from jax.experimental.pallas import tpu as pltpu
from jax.experimental.pallas import tpu_sc as plsc

N_NODES = 10000
HIDDEN = 128
N_EDGES = 320000
EDGES_IN_D = 4
NORM = 100.0

NC = 2
NS = 16
NW = NC * NS
CHUNK = 128
NCHUNKS = N_EDGES // CHUNK
CH_PER_TILE = -(-NCHUNKS // NW)
SC_NCHUNKS = NCHUNKS // NC
SC_CH_PER_TILE = -(-SC_NCHUNKS // NS)
N_NODES_PAD = 10240
ROWS_PER_TILE = N_NODES_PAD // NS
PAD = 16

_SC_MESH = plsc.VectorSubcoreMesh(core_axis_name="c", subcore_axis_name="s")

_SC_CP = pltpu.CompilerParams()
if "needs_layout_passes" in pltpu.CompilerParams.__dataclass_fields__:
    _SC_CP = dataclasses.replace(_SC_CP, needs_layout_passes=False)


GCH = 80
EDGES_PER_TILE = N_EDGES // NW
GNLOC = EDGES_PER_TILE // GCH
GNBUF = 3


@functools.partial(
    pl.kernel,
    out_type=[
        jax.ShapeDtypeStruct((N_EDGES, HIDDEN), jnp.float32),
        jax.ShapeDtypeStruct((N_EDGES, HIDDEN), jnp.float32),
    ],
    mesh=_SC_MESH,
    scratch_types=[
        pltpu.VMEM((GNBUF, GCH), jnp.int32),
        pltpu.VMEM((GNBUF, GCH), jnp.int32),
        pltpu.VMEM((GNBUF, GCH, HIDDEN), jnp.float32),
        pltpu.VMEM((GNBUF, GCH, HIDDEN), jnp.float32),
        pltpu.SemaphoreType.DMA((GNBUF,)),
        pltpu.SemaphoreType.DMA((GNBUF,)),
        pltpu.SemaphoreType.DMA((GNBUF,)),
        pltpu.SemaphoreType.DMA((GNBUF,)),
    ],
)
def _sc_gather(h_hbm, row_hbm, col_hbm, ga_hbm, gb_hbm,
               idxa, idxb, bufa, bufb, sga, sgb, soa, sob):
    wid = lax.axis_index("s") * NC + lax.axis_index("c")
    tbase = wid * EDGES_PER_TILE

    def gstart(b, j):
        base = tbase + j * GCH
        pltpu.sync_copy(row_hbm.at[pl.ds(base, GCH)], idxa.at[b])
        pltpu.sync_copy(col_hbm.at[pl.ds(base, GCH)], idxb.at[b])
        pltpu.async_copy(h_hbm.at[idxa.at[b]], bufa.at[b], sga.at[b])
        pltpu.async_copy(h_hbm.at[idxb.at[b]], bufb.at[b], sgb.at[b])

    def gwait(b):
        pltpu.make_async_copy(h_hbm.at[idxa.at[b]], bufa.at[b], sga.at[b]).wait()
        pltpu.make_async_copy(h_hbm.at[idxb.at[b]], bufb.at[b], sgb.at[b]).wait()

    def wstart(b, j):
        base = tbase + j * GCH
        pltpu.async_copy(bufa.at[b], ga_hbm.at[pl.ds(base, GCH)], soa.at[b])
        pltpu.async_copy(bufb.at[b], gb_hbm.at[pl.ds(base, GCH)], sob.at[b])

    def wwait(b):
        pltpu.make_async_copy(bufa.at[b], ga_hbm.at[pl.ds(0, GCH)], soa.at[b]).wait()
        pltpu.make_async_copy(bufb.at[b], gb_hbm.at[pl.ds(0, GCH)], sob.at[b]).wait()

    gstart(0, 0)
    gstart(1, 1)

    @pl.loop(0, (GNLOC - 2) // 3)
    def _(t):
        for u in range(3):
            j = 3 * t + u
            gwait(u)
            wstart(u, j)
            nb = (u + 2) % 3
            if u == 0:
                @pl.when(t > 0)
                def _():
                    wwait(nb)
            else:
                wwait(nb)
            gstart(nb, j + 2)

    gwait(0)
    wstart(0, GNLOC - 2)
    gwait(1)
    wstart(1, GNLOC - 1)
    wwait(2)
    wwait(0)
    wwait(1)


_BLK = 512


def _mlp_body(ga, gb, ea, w1a, w1b, w1c, b1, w2, b2, w3, out):
    x = jnp.dot(ga[...], w1a[...], preferred_element_type=jnp.float32)
    x = x + jnp.dot(gb[...], w1b[...], preferred_element_type=jnp.float32)
    x = x + jnp.dot(ea[...], w1c[...], preferred_element_type=jnp.float32)
    x = x + b1[...]
    x = x / (1.0 + jnp.exp(-x))
    x = jnp.dot(x, w2[...], preferred_element_type=jnp.float32) + b2[...]
    x = x / (1.0 + jnp.exp(-x))
    out[...] = jnp.sum(x * w3[...], axis=1)


def _tc_mlp(ga, gb, edge_attr, w1a, w1b, w1c, b1, w2, b2, w3):
    grid = (N_EDGES // _BLK,)
    full = lambda shape: pl.BlockSpec(shape, lambda i: (0, 0))
    return pl.pallas_call(
        _mlp_body,
        grid=grid,
        in_specs=[
            pl.BlockSpec((_BLK, HIDDEN), lambda i: (i, 0)),
            pl.BlockSpec((_BLK, HIDDEN), lambda i: (i, 0)),
            pl.BlockSpec((_BLK, EDGES_IN_D), lambda i: (i, 0)),
            full((HIDDEN, HIDDEN)),
            full((HIDDEN, HIDDEN)),
            full((EDGES_IN_D, HIDDEN)),
            full((1, HIDDEN)),
            full((HIDDEN, HIDDEN)),
            full((1, HIDDEN)),
            full((1, HIDDEN)),
        ],
        out_specs=pl.BlockSpec((_BLK,), lambda i: (i,)),
        out_shape=jax.ShapeDtypeStruct((N_EDGES,), jnp.float32),
    )(ga, gb, edge_attr, w1a, w1b, w1c, b1, w2, b2, w3)


SCH = 80
SC_EDGES_PER_TILE = N_EDGES // NW
SC_NLOC = SC_EDGES_PER_TILE // SCH


@functools.partial(
    pl.kernel,
    out_type=[
        jax.ShapeDtypeStruct((N_NODES_PAD, PAD), jnp.float32),
        jax.ShapeDtypeStruct((N_NODES_PAD, PAD), jnp.float32),
    ],
    mesh=_SC_MESH,
    scratch_types=[
        pltpu.VMEM((SCH,), jnp.int32),
        pltpu.VMEM((SCH,), jnp.float32),
        pltpu.VMEM((SCH,), jnp.float32),
        pltpu.VMEM((SCH,), jnp.float32),
        pltpu.VMEM((SCH,), jnp.float32),
        pltpu.VMEM((SCH, PAD), jnp.float32),
        pltpu.VMEM_SHARED((N_NODES_PAD, PAD), jnp.float32),
    ],
    compiler_params=_SC_CP,
)
def _sc_scatter(m_hbm, row_hbm, cdx_hbm, cdy_hbm, cdz_hbm, zeros_hbm,
                agg0_hbm, agg1_hbm,
                idx, m_v, cdx_v, cdy_v, cdz_v, buf, shared):
    cid = lax.axis_index("c")
    sid = lax.axis_index("s")
    rbase = sid * ROWS_PER_TILE
    tbase = (sid * NC + cid) * SC_EDGES_PER_TILE

    pltpu.sync_copy(zeros_hbm.at[pl.ds(0, SCH)], buf)
    pltpu.sync_copy(zeros_hbm.at[pl.ds(rbase, ROWS_PER_TILE)],
                    shared.at[pl.ds(rbase, ROWS_PER_TILE)])
    plsc.subcore_barrier()

    iot = lax.iota(jnp.int32, 16)

    @pl.loop(0, SC_NLOC)
    def _(j):
        base = tbase + j * SCH
        pltpu.sync_copy(row_hbm.at[pl.ds(base, SCH)], idx)
        pltpu.sync_copy(m_hbm.at[pl.ds(base, SCH)], m_v)
        pltpu.sync_copy(cdx_hbm.at[pl.ds(base, SCH)], cdx_v)
        pltpu.sync_copy(cdy_hbm.at[pl.ds(base, SCH)], cdy_v)
        pltpu.sync_copy(cdz_hbm.at[pl.ds(base, SCH)], cdz_v)
        for k in range(SCH // 16):
            rows = iot + (16 * k)
            mk = m_v[pl.ds(16 * k, 16)]
            for c, cdv in enumerate((cdx_v, cdy_v, cdz_v)):
                vals = mk * cdv[pl.ds(16 * k, 16)]
                plsc.store_scatter(
                    buf, [rows, jnp.full((16,), c, jnp.int32)], vals)
        pltpu.sync_copy(buf, shared.at[idx], add=True)

    plsc.subcore_barrier()

    @pl.when(cid == 0)
    def _():
        pltpu.sync_copy(shared.at[pl.ds(rbase, ROWS_PER_TILE)],
                        agg0_hbm.at[pl.ds(rbase, ROWS_PER_TILE)])

    @pl.when(cid == 1)
    def _():
        pltpu.sync_copy(shared.at[pl.ds(rbase, ROWS_PER_TILE)],
                        agg1_hbm.at[pl.ds(rbase, ROWS_PER_TILE)])


def _fin_body(coord, agg0, agg1, nm, out):
    s = agg0[...] + agg1[...]
    out[...] = (coord[...] + s[:N_NODES, :3] * (1.0 / NORM)) * nm[...]


def _tc_fin(coord, agg0, agg1, node_mask):
    return pl.pallas_call(
        _fin_body,
        out_shape=jax.ShapeDtypeStruct((N_NODES, 3), jnp.float32),
    )(coord, agg0, agg1, node_mask)


def kernel(h, coord, edge_index, coord_diff, edge_attr, node_mask, edge_mask,
           W1, b1, W2, b2, W3):
    row = edge_index[0].astype(jnp.int32)
    col = edge_index[1].astype(jnp.int32)

    ga, gb = _sc_gather(h, row, col)

    w1a = W1[:, :HIDDEN].T
    w1b = W1[:, HIDDEN:2 * HIDDEN].T
    w1c = W1[:, 2 * HIDDEN:].T
    m = _tc_mlp(ga, gb, edge_attr,
                w1a, w1b, w1c, b1.reshape(1, -1), W2.T, b2.reshape(1, -1),
                W3.reshape(1, -1))

    cdm = coord_diff * edge_mask
    cdx = cdm[:, 0]
    cdy = cdm[:, 1]
    cdz = cdm[:, 2]
    zeros = jnp.zeros((N_NODES_PAD, PAD), jnp.float32)
    agg0, agg1 = _sc_scatter(m, row, cdx, cdy, cdz, zeros)
    return _tc_fin(coord, agg0, agg1, node_mask)


# speedup vs baseline: 1.0989x; 1.0989x over previous
import dataclasses
import functools

import jax
import jax.numpy as jnp
from jax import lax
from jax.experimental import pallas as pl
from jax.experimental.pallas import tpu as pltpu
from jax.experimental.pallas import tpu_sc as plsc

N_NODES = 10000
HIDDEN = 128
N_EDGES = 320000
EDGES_IN_D = 4
NORM = 100.0

NC = 2
NS = 16
NW = NC * NS
CHUNK = 128
NCHUNKS = N_EDGES // CHUNK
CH_PER_TILE = -(-NCHUNKS // NW)
SC_NCHUNKS = NCHUNKS // NC
SC_CH_PER_TILE = -(-SC_NCHUNKS // NS)
N_NODES_PAD = 10240
ROWS_PER_TILE = N_NODES_PAD // NS
PAD = 16

_SC_MESH = plsc.VectorSubcoreMesh(core_axis_name="c", subcore_axis_name="s")

_SC_CP = pltpu.CompilerParams()
if "needs_layout_passes" in pltpu.CompilerParams.__dataclass_fields__:
    _SC_CP = dataclasses.replace(_SC_CP, needs_layout_passes=False)


GCH = 80
EDGES_PER_TILE = N_EDGES // NW
GNLOC = EDGES_PER_TILE // GCH
GNBUF = 3


@functools.partial(
    pl.kernel,
    out_type=[
        jax.ShapeDtypeStruct((N_EDGES, HIDDEN), jnp.float32),
        jax.ShapeDtypeStruct((N_EDGES, HIDDEN), jnp.float32),
    ],
    mesh=_SC_MESH,
    scratch_types=[
        pltpu.VMEM((GNBUF, GCH), jnp.int32),
        pltpu.VMEM((GNBUF, GCH), jnp.int32),
        pltpu.VMEM((GNBUF, GCH, HIDDEN), jnp.float32),
        pltpu.VMEM((GNBUF, GCH, HIDDEN), jnp.float32),
        pltpu.SemaphoreType.DMA((GNBUF,)),
        pltpu.SemaphoreType.DMA((GNBUF,)),
        pltpu.SemaphoreType.DMA((GNBUF,)),
        pltpu.SemaphoreType.DMA((GNBUF,)),
    ],
)
def _sc_gather(h_hbm, row_hbm, col_hbm, ga_hbm, gb_hbm,
               idxa, idxb, bufa, bufb, sga, sgb, soa, sob):
    wid = lax.axis_index("s") * NC + lax.axis_index("c")
    tbase = wid * EDGES_PER_TILE

    def gstart(b, j):
        base = tbase + j * GCH
        pltpu.sync_copy(row_hbm.at[pl.ds(base, GCH)], idxa.at[b])
        pltpu.sync_copy(col_hbm.at[pl.ds(base, GCH)], idxb.at[b])
        pltpu.async_copy(h_hbm.at[idxa.at[b]], bufa.at[b], sga.at[b])
        pltpu.async_copy(h_hbm.at[idxb.at[b]], bufb.at[b], sgb.at[b])

    def gwait(b):
        pltpu.make_async_copy(h_hbm.at[idxa.at[b]], bufa.at[b], sga.at[b]).wait()
        pltpu.make_async_copy(h_hbm.at[idxb.at[b]], bufb.at[b], sgb.at[b]).wait()

    def wstart(b, j):
        base = tbase + j * GCH
        pltpu.async_copy(bufa.at[b], ga_hbm.at[pl.ds(base, GCH)], soa.at[b])
        pltpu.async_copy(bufb.at[b], gb_hbm.at[pl.ds(base, GCH)], sob.at[b])

    def wwait(b):
        pltpu.make_async_copy(bufa.at[b], ga_hbm.at[pl.ds(0, GCH)], soa.at[b]).wait()
        pltpu.make_async_copy(bufb.at[b], gb_hbm.at[pl.ds(0, GCH)], sob.at[b]).wait()

    gstart(0, 0)
    gstart(1, 1)

    @pl.loop(0, (GNLOC - 2) // 3)
    def _(t):
        for u in range(3):
            j = 3 * t + u
            gwait(u)
            wstart(u, j)
            nb = (u + 2) % 3
            if u == 0:
                @pl.when(t > 0)
                def _():
                    wwait(nb)
            else:
                wwait(nb)
            gstart(nb, j + 2)

    gwait(0)
    wstart(0, GNLOC - 2)
    gwait(1)
    wstart(1, GNLOC - 1)
    wwait(2)
    wwait(0)
    wwait(1)


_BLK = 2000
_NBLK = N_EDGES // _BLK


def _mlp_body(ga, gb, ea, w1a, w1b, w1c, b1, w2, b2, w3, out):
    x = jnp.dot(ga[...], w1a[...], preferred_element_type=jnp.float32)
    x = x + jnp.dot(gb[...], w1b[...], preferred_element_type=jnp.float32)
    x = x + jnp.dot(ea[...], w1c[...], preferred_element_type=jnp.float32)
    x = x + b1[...]
    x = x / (1.0 + jnp.exp(-x))
    x = jnp.dot(x, w2[...], preferred_element_type=jnp.float32) + b2[...]
    x = x / (1.0 + jnp.exp(-x))
    out[...] = jnp.sum(x * w3[...], axis=1).reshape(1, 1, _BLK)


def _tc_mlp(ga, gb, edge_attr, w1a, w1b, w1c, b1, w2, b2, w3):
    grid = (N_EDGES // _BLK,)
    full = lambda shape: pl.BlockSpec(shape, lambda i: (0, 0))
    return pl.pallas_call(
        _mlp_body,
        grid=grid,
        in_specs=[
            pl.BlockSpec((_BLK, HIDDEN), lambda i: (i, 0)),
            pl.BlockSpec((_BLK, HIDDEN), lambda i: (i, 0)),
            pl.BlockSpec((_BLK, EDGES_IN_D), lambda i: (i, 0)),
            full((HIDDEN, HIDDEN)),
            full((HIDDEN, HIDDEN)),
            full((EDGES_IN_D, HIDDEN)),
            full((1, HIDDEN)),
            full((HIDDEN, HIDDEN)),
            full((1, HIDDEN)),
            full((1, HIDDEN)),
        ],
        out_specs=pl.BlockSpec((1, 1, _BLK), lambda i: (i, 0, 0)),
        out_shape=jax.ShapeDtypeStruct((_NBLK, 1, _BLK), jnp.float32),
    )(ga, gb, edge_attr, w1a, w1b, w1c, b1, w2, b2, w3)


SCH = 80
SC_EDGES_PER_TILE = N_EDGES // NW
SC_NLOC = SC_EDGES_PER_TILE // SCH


@functools.partial(
    pl.kernel,
    out_type=[
        jax.ShapeDtypeStruct((N_NODES_PAD, PAD), jnp.float32),
        jax.ShapeDtypeStruct((N_NODES_PAD, PAD), jnp.float32),
    ],
    mesh=_SC_MESH,
    scratch_types=[
        pltpu.VMEM((SCH,), jnp.float32),
        pltpu.VMEM((SCH,), jnp.float32),
        pltpu.VMEM((SCH,), jnp.float32),
        pltpu.VMEM((SCH,), jnp.float32),
        pltpu.VMEM((2, SCH), jnp.int32),
        pltpu.VMEM((SCH, PAD), jnp.float32),
        pltpu.VMEM_SHARED((N_NODES_PAD, PAD), jnp.float32),
        pltpu.SemaphoreType.DMA((2,)),
    ],
    compiler_params=_SC_CP,
)
def _sc_scatter(m_hbm, row_hbm, cdx_hbm, cdy_hbm, cdz_hbm, zeros_hbm,
                agg0_hbm, agg1_hbm,
                m_v, cdx_v, cdy_v, cdz_v, idx2, buf, shared, semi):
    cid = lax.axis_index("c")
    sid = lax.axis_index("s")
    rbase = sid * ROWS_PER_TILE
    tbase = (sid * NC + cid) * SC_EDGES_PER_TILE

    def pref(b, j):
        pltpu.async_copy(row_hbm.at[pl.ds(tbase + j * SCH, SCH)],
                         idx2.at[b], semi.at[b])

    def prefwait(b):
        pltpu.make_async_copy(row_hbm.at[pl.ds(tbase, SCH)],
                              idx2.at[b], semi.at[b]).wait()

    pref(0, 0)
    pltpu.sync_copy(zeros_hbm.at[pl.ds(0, SCH)], buf)
    pltpu.sync_copy(zeros_hbm.at[pl.ds(rbase, ROWS_PER_TILE)],
                    shared.at[pl.ds(rbase, ROWS_PER_TILE)])
    plsc.subcore_barrier()

    iot = lax.iota(jnp.int32, 16)

    def body(b, j, do_pref):
        prefwait(b)
        if do_pref:
            pref(1 - b, j + 1)
        off = j * SCH
        pltpu.sync_copy(m_hbm.at[pl.ds(tbase + off, SCH)], m_v)
        pltpu.sync_copy(cdx_hbm.at[pl.ds(tbase + off, SCH)], cdx_v)
        pltpu.sync_copy(cdy_hbm.at[pl.ds(tbase + off, SCH)], cdy_v)
        pltpu.sync_copy(cdz_hbm.at[pl.ds(tbase + off, SCH)], cdz_v)
        for k in range(SCH // 16):
            rows = iot + (16 * k)
            mk = m_v[pl.ds(16 * k, 16)]
            for c, cdv in enumerate((cdx_v, cdy_v, cdz_v)):
                vals = mk * cdv[pl.ds(16 * k, 16)]
                plsc.store_scatter(
                    buf, [rows, jnp.full((16,), c, jnp.int32)], vals)
        pltpu.sync_copy(buf, shared.at[idx2.at[b]], add=True)

    @pl.loop(0, (SC_NLOC - 1) // 2)
    def _(t):
        body(0, 2 * t, True)
        body(1, 2 * t + 1, True)

    body(0, SC_NLOC - 1, False)
    plsc.subcore_barrier()

    @pl.when(cid == 0)
    def _():
        pltpu.sync_copy(shared.at[pl.ds(rbase, ROWS_PER_TILE)],
                        agg0_hbm.at[pl.ds(rbase, ROWS_PER_TILE)])

    @pl.when(cid == 1)
    def _():
        pltpu.sync_copy(shared.at[pl.ds(rbase, ROWS_PER_TILE)],
                        agg1_hbm.at[pl.ds(rbase, ROWS_PER_TILE)])


def _fin_body(coord, agg0, agg1, nm, out):
    s = agg0[...] + agg1[...]
    out[...] = (coord[...] + s[:N_NODES, :3] * (1.0 / NORM)) * nm[...]


def _tc_fin(coord, agg0, agg1, node_mask):
    return pl.pallas_call(
        _fin_body,
        out_shape=jax.ShapeDtypeStruct((N_NODES, 3), jnp.float32),
    )(coord, agg0, agg1, node_mask)


def kernel(h, coord, edge_index, coord_diff, edge_attr, node_mask, edge_mask,
           W1, b1, W2, b2, W3):
    row = edge_index[0].astype(jnp.int32)
    col = edge_index[1].astype(jnp.int32)

    ga, gb = _sc_gather(h, row, col)

    w1a = W1[:, :HIDDEN].T
    w1b = W1[:, HIDDEN:2 * HIDDEN].T
    w1c = W1[:, 2 * HIDDEN:].T
    m = _tc_mlp(ga, gb, edge_attr,
                w1a, w1b, w1c, b1.reshape(1, -1), W2.T, b2.reshape(1, -1),
                W3.reshape(1, -1))

    cdm = coord_diff * edge_mask
    cdx = cdm[:, 0]
    cdy = cdm[:, 1]
    cdz = cdm[:, 2]
    zeros = jnp.zeros((N_NODES_PAD, PAD), jnp.float32)
    agg0, agg1 = _sc_scatter(m.reshape(N_EDGES), row, cdx, cdy, cdz, zeros)
    return _tc_fin(coord, agg0, agg1, node_mask)


# speedup vs baseline: 1.4657x; 1.3338x over previous
import dataclasses
import functools

import jax
import jax.numpy as jnp
from jax import lax
from jax.experimental import pallas as pl
from jax.experimental.pallas import tpu as pltpu
from jax.experimental.pallas import tpu_sc as plsc

N_NODES = 10000
HIDDEN = 128
N_EDGES = 320000
EDGES_IN_D = 4
NORM = 100.0

NC = 2
NS = 16
NW = NC * NS
CHUNK = 128
NCHUNKS = N_EDGES // CHUNK
CH_PER_TILE = -(-NCHUNKS // NW)
SC_NCHUNKS = NCHUNKS // NC
SC_CH_PER_TILE = -(-SC_NCHUNKS // NS)
N_NODES_PAD = 10240
ROWS_PER_TILE = N_NODES_PAD // NS
PAD = 16

_SC_MESH = plsc.VectorSubcoreMesh(core_axis_name="c", subcore_axis_name="s")

_SC_CP = pltpu.CompilerParams()
if "needs_layout_passes" in pltpu.CompilerParams.__dataclass_fields__:
    _SC_CP = dataclasses.replace(_SC_CP, needs_layout_passes=False)


GCH = 80
EDGES_PER_TILE = N_EDGES // NW
GNLOC = EDGES_PER_TILE // GCH
GNBUF = 3


@functools.partial(
    pl.kernel,
    out_type=[
        jax.ShapeDtypeStruct((N_EDGES, HIDDEN), jnp.float32),
        jax.ShapeDtypeStruct((N_EDGES, HIDDEN), jnp.float32),
    ],
    mesh=_SC_MESH,
    scratch_types=[
        pltpu.VMEM((GNBUF, GCH), jnp.int32),
        pltpu.VMEM((GNBUF, GCH), jnp.int32),
        pltpu.VMEM((GNBUF, GCH, HIDDEN), jnp.float32),
        pltpu.VMEM((GNBUF, GCH, HIDDEN), jnp.float32),
        pltpu.SemaphoreType.DMA((GNBUF,)),
        pltpu.SemaphoreType.DMA((GNBUF,)),
        pltpu.SemaphoreType.DMA((GNBUF,)),
        pltpu.SemaphoreType.DMA((GNBUF,)),
    ],
)
def _sc_gather(h_hbm, row_hbm, col_hbm, ga_hbm, gb_hbm,
               idxa, idxb, bufa, bufb, sga, sgb, soa, sob):
    wid = lax.axis_index("s") * NC + lax.axis_index("c")
    tbase = wid * EDGES_PER_TILE

    def gstart(b, j):
        base = tbase + j * GCH
        pltpu.sync_copy(row_hbm.at[pl.ds(base, GCH)], idxa.at[b])
        pltpu.sync_copy(col_hbm.at[pl.ds(base, GCH)], idxb.at[b])
        pltpu.async_copy(h_hbm.at[idxa.at[b]], bufa.at[b], sga.at[b])
        pltpu.async_copy(h_hbm.at[idxb.at[b]], bufb.at[b], sgb.at[b])

    def gwait(b):
        pltpu.make_async_copy(h_hbm.at[idxa.at[b]], bufa.at[b], sga.at[b]).wait()
        pltpu.make_async_copy(h_hbm.at[idxb.at[b]], bufb.at[b], sgb.at[b]).wait()

    def wstart(b, j):
        base = tbase + j * GCH
        pltpu.async_copy(bufa.at[b], ga_hbm.at[pl.ds(base, GCH)], soa.at[b])
        pltpu.async_copy(bufb.at[b], gb_hbm.at[pl.ds(base, GCH)], sob.at[b])

    def wwait(b):
        pltpu.make_async_copy(bufa.at[b], ga_hbm.at[pl.ds(0, GCH)], soa.at[b]).wait()
        pltpu.make_async_copy(bufb.at[b], gb_hbm.at[pl.ds(0, GCH)], sob.at[b]).wait()

    gstart(0, 0)
    gstart(1, 1)

    @pl.loop(0, (GNLOC - 2) // 3)
    def _(t):
        for u in range(3):
            j = 3 * t + u
            gwait(u)
            wstart(u, j)
            nb = (u + 2) % 3
            if u == 0:
                @pl.when(t > 0)
                def _():
                    wwait(nb)
            else:
                wwait(nb)
            gstart(nb, j + 2)

    gwait(0)
    wstart(0, GNLOC - 2)
    gwait(1)
    wstart(1, GNLOC - 1)
    wwait(2)
    wwait(0)
    wwait(1)


_BLK = 2000
_NBLK = N_EDGES // _BLK


def _mlp_body(ga, gb, ea, w1a, w1b, w1c, b1, w2, b2, w3, out):
    x = jnp.dot(ga[...], w1a[...], preferred_element_type=jnp.float32)
    x = x + jnp.dot(gb[...], w1b[...], preferred_element_type=jnp.float32)
    x = x + jnp.dot(ea[...], w1c[...], preferred_element_type=jnp.float32)
    x = x + b1[...]
    x = x / (1.0 + jnp.exp(-x))
    x = jnp.dot(x, w2[...], preferred_element_type=jnp.float32) + b2[...]
    x = x / (1.0 + jnp.exp(-x))
    m = jax.lax.dot_general(w3[...], x, (((1,), (1,)), ((), ())),
                            preferred_element_type=jnp.float32)
    out[...] = m.reshape(1, 1, _BLK)


def _tc_mlp(ga, gb, edge_attr, w1a, w1b, w1c, b1, w2, b2, w3):
    grid = (N_EDGES // _BLK,)
    full = lambda shape: pl.BlockSpec(shape, lambda i: (0, 0))
    return pl.pallas_call(
        _mlp_body,
        grid=grid,
        in_specs=[
            pl.BlockSpec((_BLK, HIDDEN), lambda i: (i, 0)),
            pl.BlockSpec((_BLK, HIDDEN), lambda i: (i, 0)),
            pl.BlockSpec((_BLK, EDGES_IN_D), lambda i: (i, 0)),
            full((HIDDEN, HIDDEN)),
            full((HIDDEN, HIDDEN)),
            full((EDGES_IN_D, HIDDEN)),
            full((1, HIDDEN)),
            full((HIDDEN, HIDDEN)),
            full((1, HIDDEN)),
            full((1, HIDDEN)),
        ],
        out_specs=pl.BlockSpec((1, 1, _BLK), lambda i: (i, 0, 0)),
        out_shape=jax.ShapeDtypeStruct((_NBLK, 1, _BLK), jnp.float32),
    )(ga, gb, edge_attr, w1a, w1b, w1c, b1, w2, b2, w3)


SCH = 80
SC_EDGES_PER_TILE = N_EDGES // NW
SC_NLOC = SC_EDGES_PER_TILE // SCH


@functools.partial(
    pl.kernel,
    out_type=[
        jax.ShapeDtypeStruct((N_NODES_PAD, PAD), jnp.float32),
        jax.ShapeDtypeStruct((N_NODES_PAD, PAD), jnp.float32),
    ],
    mesh=_SC_MESH,
    scratch_types=[
        pltpu.VMEM((SCH,), jnp.float32),
        pltpu.VMEM((SCH,), jnp.float32),
        pltpu.VMEM((SCH,), jnp.float32),
        pltpu.VMEM((SCH,), jnp.float32),
        pltpu.VMEM((2, SCH), jnp.int32),
        pltpu.VMEM((SCH, PAD), jnp.float32),
        pltpu.VMEM_SHARED((N_NODES_PAD, PAD), jnp.float32),
        pltpu.SemaphoreType.DMA((2,)),
    ],
    compiler_params=_SC_CP,
)
def _sc_scatter(m_hbm, row_hbm, cdx_hbm, cdy_hbm, cdz_hbm, zeros_hbm,
                agg0_hbm, agg1_hbm,
                m_v, cdx_v, cdy_v, cdz_v, idx2, buf, shared, semi):
    cid = lax.axis_index("c")
    sid = lax.axis_index("s")
    rbase = sid * ROWS_PER_TILE
    tbase = (sid * NC + cid) * SC_EDGES_PER_TILE

    def pref(b, j):
        pltpu.async_copy(row_hbm.at[pl.ds(tbase + j * SCH, SCH)],
                         idx2.at[b], semi.at[b])

    def prefwait(b):
        pltpu.make_async_copy(row_hbm.at[pl.ds(tbase, SCH)],
                              idx2.at[b], semi.at[b]).wait()

    pref(0, 0)
    pltpu.sync_copy(zeros_hbm.at[pl.ds(0, SCH)], buf)
    pltpu.sync_copy(zeros_hbm.at[pl.ds(rbase, ROWS_PER_TILE)],
                    shared.at[pl.ds(rbase, ROWS_PER_TILE)])
    plsc.subcore_barrier()

    iot = lax.iota(jnp.int32, 16)

    def body(b, j, do_pref):
        prefwait(b)
        if do_pref:
            pref(1 - b, j + 1)
        off = j * SCH
        pltpu.sync_copy(m_hbm.at[pl.ds(tbase + off, SCH)], m_v)
        pltpu.sync_copy(cdx_hbm.at[pl.ds(tbase + off, SCH)], cdx_v)
        pltpu.sync_copy(cdy_hbm.at[pl.ds(tbase + off, SCH)], cdy_v)
        pltpu.sync_copy(cdz_hbm.at[pl.ds(tbase + off, SCH)], cdz_v)
        for k in range(SCH // 16):
            rows = iot + (16 * k)
            mk = m_v[pl.ds(16 * k, 16)]
            for c, cdv in enumerate((cdx_v, cdy_v, cdz_v)):
                vals = mk * cdv[pl.ds(16 * k, 16)]
                plsc.store_scatter(
                    buf, [rows, jnp.full((16,), c, jnp.int32)], vals)
        pltpu.sync_copy(buf, shared.at[idx2.at[b]], add=True)

    @pl.loop(0, (SC_NLOC - 1) // 2)
    def _(t):
        body(0, 2 * t, True)
        body(1, 2 * t + 1, True)

    body(0, SC_NLOC - 1, False)
    plsc.subcore_barrier()

    @pl.when(cid == 0)
    def _():
        pltpu.sync_copy(shared.at[pl.ds(rbase, ROWS_PER_TILE)],
                        agg0_hbm.at[pl.ds(rbase, ROWS_PER_TILE)])

    @pl.when(cid == 1)
    def _():
        pltpu.sync_copy(shared.at[pl.ds(rbase, ROWS_PER_TILE)],
                        agg1_hbm.at[pl.ds(rbase, ROWS_PER_TILE)])


def _fin_body(coord, agg0, agg1, nm, out):
    s = agg0[...] + agg1[...]
    out[...] = (coord[...] + s[:N_NODES, :3] * (1.0 / NORM)) * nm[...]


def _tc_fin(coord, agg0, agg1, node_mask):
    return pl.pallas_call(
        _fin_body,
        out_shape=jax.ShapeDtypeStruct((N_NODES, 3), jnp.float32),
    )(coord, agg0, agg1, node_mask)


def kernel(h, coord, edge_index, coord_diff, edge_attr, node_mask, edge_mask,
           W1, b1, W2, b2, W3):
    row = edge_index[0].astype(jnp.int32)
    col = edge_index[1].astype(jnp.int32)

    ga, gb = _sc_gather(h, row, col)

    w1a = W1[:, :HIDDEN].T
    w1b = W1[:, HIDDEN:2 * HIDDEN].T
    w1c = W1[:, 2 * HIDDEN:].T
    m = _tc_mlp(ga, gb, edge_attr,
                w1a, w1b, w1c, b1.reshape(1, -1), W2.T, b2.reshape(1, -1),
                W3.reshape(1, -1))

    cdm = coord_diff * edge_mask
    cdx = cdm[:, 0]
    cdy = cdm[:, 1]
    cdz = cdm[:, 2]
    zeros = jnp.zeros((N_NODES_PAD, PAD), jnp.float32)
    agg0, agg1 = _sc_scatter(m.reshape(N_EDGES), row, cdx, cdy, cdz, zeros)
    return _tc_fin(coord, agg0, agg1, node_mask)


# speedup vs baseline: 1.8489x; 1.2614x over previous
import dataclasses
import functools

import jax
import jax.numpy as jnp
from jax import lax
from jax.experimental import pallas as pl
from jax.experimental.pallas import tpu as pltpu
from jax.experimental.pallas import tpu_sc as plsc

N_NODES = 10000
HIDDEN = 128
N_EDGES = 320000
EDGES_IN_D = 4
NORM = 100.0

NC = 2
NS = 16
NW = NC * NS
CHUNK = 128
NCHUNKS = N_EDGES // CHUNK
CH_PER_TILE = -(-NCHUNKS // NW)
SC_NCHUNKS = NCHUNKS // NC
SC_CH_PER_TILE = -(-SC_NCHUNKS // NS)
N_NODES_PAD = 10240
ROWS_PER_TILE = N_NODES_PAD // NS
PAD = 16

_SC_MESH = plsc.VectorSubcoreMesh(core_axis_name="c", subcore_axis_name="s")

_SC_CP = pltpu.CompilerParams()
if "needs_layout_passes" in pltpu.CompilerParams.__dataclass_fields__:
    _SC_CP = dataclasses.replace(_SC_CP, needs_layout_passes=False)


GCH = 80
EDGES_PER_TILE = N_EDGES // NW
GNLOC = EDGES_PER_TILE // GCH
GNBUF = 3


@functools.partial(
    pl.kernel,
    out_type=[
        jax.ShapeDtypeStruct((N_EDGES, HIDDEN), jnp.float32),
        jax.ShapeDtypeStruct((N_EDGES, HIDDEN), jnp.float32),
    ],
    mesh=_SC_MESH,
    scratch_types=[
        pltpu.VMEM((GNBUF, GCH), jnp.int32),
        pltpu.VMEM((GNBUF, GCH), jnp.int32),
        pltpu.VMEM((GNBUF, GCH, HIDDEN), jnp.float32),
        pltpu.VMEM((GNBUF, GCH, HIDDEN), jnp.float32),
        pltpu.SemaphoreType.DMA((GNBUF,)),
        pltpu.SemaphoreType.DMA((GNBUF,)),
        pltpu.SemaphoreType.DMA((GNBUF,)),
        pltpu.SemaphoreType.DMA((GNBUF,)),
    ],
)
def _sc_gather(h_hbm, row_hbm, col_hbm, ga_hbm, gb_hbm,
               idxa, idxb, bufa, bufb, sga, sgb, soa, sob):
    wid = lax.axis_index("s") * NC + lax.axis_index("c")
    tbase = wid * EDGES_PER_TILE

    def gstart(b, j):
        base = tbase + j * GCH
        pltpu.sync_copy(row_hbm.at[pl.ds(base, GCH)], idxa.at[b])
        pltpu.sync_copy(col_hbm.at[pl.ds(base, GCH)], idxb.at[b])
        pltpu.async_copy(h_hbm.at[idxa.at[b]], bufa.at[b], sga.at[b])
        pltpu.async_copy(h_hbm.at[idxb.at[b]], bufb.at[b], sgb.at[b])

    def gwait(b):
        pltpu.make_async_copy(h_hbm.at[idxa.at[b]], bufa.at[b], sga.at[b]).wait()
        pltpu.make_async_copy(h_hbm.at[idxb.at[b]], bufb.at[b], sgb.at[b]).wait()

    def wstart(b, j):
        base = tbase + j * GCH
        pltpu.async_copy(bufa.at[b], ga_hbm.at[pl.ds(base, GCH)], soa.at[b])
        pltpu.async_copy(bufb.at[b], gb_hbm.at[pl.ds(base, GCH)], sob.at[b])

    def wwait(b):
        pltpu.make_async_copy(bufa.at[b], ga_hbm.at[pl.ds(0, GCH)], soa.at[b]).wait()
        pltpu.make_async_copy(bufb.at[b], gb_hbm.at[pl.ds(0, GCH)], sob.at[b]).wait()

    gstart(0, 0)
    gstart(1, 1)

    @pl.loop(0, (GNLOC - 2) // 3)
    def _(t):
        for u in range(3):
            j = 3 * t + u
            gwait(u)
            wstart(u, j)
            nb = (u + 2) % 3
            if u == 0:
                @pl.when(t > 0)
                def _():
                    wwait(nb)
            else:
                wwait(nb)
            gstart(nb, j + 2)

    gwait(0)
    wstart(0, GNLOC - 2)
    gwait(1)
    wstart(1, GNLOC - 1)
    wwait(2)
    wwait(0)
    wwait(1)


_BLK = 2000
_NBLK = N_EDGES // _BLK


def _mlp_body(ga, gb, ea, w1a, w1b, w1c, b1, w2, b2, w3, out):
    x = jnp.dot(ga[...], w1a[...], preferred_element_type=jnp.float32)
    x = x + jnp.dot(gb[...], w1b[...], preferred_element_type=jnp.float32)
    x = x + jnp.dot(ea[...], w1c[...], preferred_element_type=jnp.float32)
    x = x + b1[...]
    x = x / (1.0 + jnp.exp(-x))
    x = jnp.dot(x, w2[...], preferred_element_type=jnp.float32) + b2[...]
    x = x / (1.0 + jnp.exp(-x))
    m = jax.lax.dot_general(w3[...], x, (((1,), (1,)), ((), ())),
                            preferred_element_type=jnp.float32)
    out[...] = m.reshape(1, 1, _BLK)


def _tc_mlp(ga, gb, edge_attr, w1a, w1b, w1c, b1, w2, b2, w3):
    grid = (N_EDGES // _BLK,)
    full = lambda shape: pl.BlockSpec(shape, lambda i: (0, 0))
    return pl.pallas_call(
        _mlp_body,
        grid=grid,
        in_specs=[
            pl.BlockSpec((_BLK, HIDDEN), lambda i: (i, 0)),
            pl.BlockSpec((_BLK, HIDDEN), lambda i: (i, 0)),
            pl.BlockSpec((_BLK, EDGES_IN_D), lambda i: (i, 0)),
            full((HIDDEN, HIDDEN)),
            full((HIDDEN, HIDDEN)),
            full((EDGES_IN_D, HIDDEN)),
            full((1, HIDDEN)),
            full((HIDDEN, HIDDEN)),
            full((1, HIDDEN)),
            full((1, HIDDEN)),
        ],
        out_specs=pl.BlockSpec((1, 1, _BLK), lambda i: (i, 0, 0)),
        out_shape=jax.ShapeDtypeStruct((_NBLK, 1, _BLK), jnp.float32),
    )(ga, gb, edge_attr, w1a, w1b, w1c, b1, w2, b2, w3)


SCH = 80
SC_EDGES_PER_TILE = N_EDGES // NW
SC_NLOC = SC_EDGES_PER_TILE // SCH


@functools.partial(
    pl.kernel,
    out_type=[
        jax.ShapeDtypeStruct((N_NODES_PAD, PAD), jnp.float32),
        jax.ShapeDtypeStruct((N_NODES_PAD, PAD), jnp.float32),
    ],
    mesh=_SC_MESH,
    scratch_types=[
        pltpu.VMEM((2, SCH), jnp.float32),
        pltpu.VMEM((2, SCH), jnp.float32),
        pltpu.VMEM((2, SCH), jnp.float32),
        pltpu.VMEM((2, SCH), jnp.float32),
        pltpu.VMEM((2, SCH), jnp.int32),
        pltpu.VMEM((SCH, PAD), jnp.float32),
        pltpu.VMEM_SHARED((N_NODES_PAD, PAD), jnp.float32),
        pltpu.SemaphoreType.DMA((2,)),
        pltpu.SemaphoreType.DMA((2,)),
    ],
    compiler_params=_SC_CP,
)
def _sc_scatter(m_hbm, row_hbm, cdx_hbm, cdy_hbm, cdz_hbm, zeros_hbm,
                agg0_hbm, agg1_hbm,
                m_v, cdx_v, cdy_v, cdz_v, idx2, buf, shared, semi, semv):
    cid = lax.axis_index("c")
    sid = lax.axis_index("s")
    rbase = sid * ROWS_PER_TILE
    tbase = (sid * NC + cid) * SC_EDGES_PER_TILE

    def pref(b, j):
        base = tbase + j * SCH
        pltpu.async_copy(row_hbm.at[pl.ds(base, SCH)], idx2.at[b], semi.at[b])
        pltpu.async_copy(m_hbm.at[pl.ds(base, SCH)], m_v.at[b], semv.at[b])
        pltpu.async_copy(cdx_hbm.at[pl.ds(base, SCH)], cdx_v.at[b], semv.at[b])
        pltpu.async_copy(cdy_hbm.at[pl.ds(base, SCH)], cdy_v.at[b], semv.at[b])
        pltpu.async_copy(cdz_hbm.at[pl.ds(base, SCH)], cdz_v.at[b], semv.at[b])

    def prefwait(b):
        pltpu.make_async_copy(row_hbm.at[pl.ds(tbase, SCH)],
                              idx2.at[b], semi.at[b]).wait()
        pltpu.make_async_copy(m_hbm.at[pl.ds(tbase, SCH)],
                              m_v.at[b], semv.at[b]).wait()
        pltpu.make_async_copy(cdx_hbm.at[pl.ds(tbase, SCH)],
                              cdx_v.at[b], semv.at[b]).wait()
        pltpu.make_async_copy(cdy_hbm.at[pl.ds(tbase, SCH)],
                              cdy_v.at[b], semv.at[b]).wait()
        pltpu.make_async_copy(cdz_hbm.at[pl.ds(tbase, SCH)],
                              cdz_v.at[b], semv.at[b]).wait()

    pref(0, 0)
    pltpu.sync_copy(zeros_hbm.at[pl.ds(0, SCH)], buf)
    pltpu.sync_copy(zeros_hbm.at[pl.ds(rbase, ROWS_PER_TILE)],
                    shared.at[pl.ds(rbase, ROWS_PER_TILE)])
    plsc.subcore_barrier()

    iot = lax.iota(jnp.int32, 16)

    def body(b, j, do_pref):
        prefwait(b)
        if do_pref:
            pref(1 - b, j + 1)
        for k in range(SCH // 16):
            rows = iot + (16 * k)
            mk = m_v[b, pl.ds(16 * k, 16)]
            for c, cdv in enumerate((cdx_v, cdy_v, cdz_v)):
                vals = mk * cdv[b, pl.ds(16 * k, 16)]
                plsc.store_scatter(
                    buf, [rows, jnp.full((16,), c, jnp.int32)], vals)
        pltpu.sync_copy(buf, shared.at[idx2.at[b]], add=True)

    @pl.loop(0, (SC_NLOC - 1) // 2)
    def _(t):
        body(0, 2 * t, True)
        body(1, 2 * t + 1, True)

    body(0, SC_NLOC - 1, False)
    plsc.subcore_barrier()

    @pl.when(cid == 0)
    def _():
        pltpu.sync_copy(shared.at[pl.ds(rbase, ROWS_PER_TILE)],
                        agg0_hbm.at[pl.ds(rbase, ROWS_PER_TILE)])

    @pl.when(cid == 1)
    def _():
        pltpu.sync_copy(shared.at[pl.ds(rbase, ROWS_PER_TILE)],
                        agg1_hbm.at[pl.ds(rbase, ROWS_PER_TILE)])


def _fin_body(coord, agg0, agg1, nm, out):
    s = agg0[...] + agg1[...]
    out[...] = (coord[...] + s[:N_NODES, :3] * (1.0 / NORM)) * nm[...]


def _tc_fin(coord, agg0, agg1, node_mask):
    return pl.pallas_call(
        _fin_body,
        out_shape=jax.ShapeDtypeStruct((N_NODES, 3), jnp.float32),
    )(coord, agg0, agg1, node_mask)


def kernel(h, coord, edge_index, coord_diff, edge_attr, node_mask, edge_mask,
           W1, b1, W2, b2, W3):
    row = edge_index[0].astype(jnp.int32)
    col = edge_index[1].astype(jnp.int32)

    ga, gb = _sc_gather(h, row, col)

    w1a = W1[:, :HIDDEN].T
    w1b = W1[:, HIDDEN:2 * HIDDEN].T
    w1c = W1[:, 2 * HIDDEN:].T
    m = _tc_mlp(ga, gb, edge_attr,
                w1a, w1b, w1c, b1.reshape(1, -1), W2.T, b2.reshape(1, -1),
                W3.reshape(1, -1))

    cdm = coord_diff * edge_mask
    cdx = cdm[:, 0]
    cdy = cdm[:, 1]
    cdz = cdm[:, 2]
    zeros = jnp.zeros((N_NODES_PAD, PAD), jnp.float32)
    agg0, agg1 = _sc_scatter(m.reshape(N_EDGES), row, cdx, cdy, cdz, zeros)
    return _tc_fin(coord, agg0, agg1, node_mask)
